# Initial kernel scaffold; baseline (speedup 1.0000x reference)
import jax
import jax.numpy as jnp
from jax.experimental import pallas as pl


def kernel(users, items, user_table, item_table):
    raise NotImplementedError("write your pallas kernel here")


# speedup vs baseline: 1.5606x; 1.5606x over previous
import functools

import jax
import jax.numpy as jnp
from jax import lax
from jax.experimental import pallas as pl
from jax.experimental.pallas import tpu as pltpu
from jax.experimental.pallas import tpu_sc as plsc

BATCH = 4096
DIM = 4096
NUM_CORES = 2
NUM_SUBCORES = 16
NUM_WORKERS = NUM_CORES * NUM_SUBCORES
BPW = BATCH // NUM_WORKERS
CHUNK = 8
NCHUNK = BPW // CHUNK

_MESH = plsc.VectorSubcoreMesh(
    core_axis_name="c", subcore_axis_name="s",
    num_cores=NUM_CORES, num_subcores=NUM_SUBCORES)


@functools.partial(
    pl.kernel,
    out_type=(
        jax.ShapeDtypeStruct((BATCH, DIM), jnp.float32),
        jax.ShapeDtypeStruct((BATCH, DIM), jnp.float32),
    ),
    mesh=_MESH,
    scratch_types=[
        pltpu.VMEM((BPW,), jnp.int32),
        pltpu.VMEM((BPW,), jnp.int32),
        pltpu.VMEM((CHUNK, DIM), jnp.float32),
        pltpu.SemaphoreType.DMA,
    ],
)
def _gather2(users_hbm, items_hbm, u_tab, i_tab, u_out, i_out,
             uidx, iidx, buf, gsem):
    wid = lax.axis_index("s") * NUM_CORES + lax.axis_index("c")
    base = wid * BPW
    pltpu.sync_copy(users_hbm.at[pl.ds(base, BPW)], uidx)
    pltpu.sync_copy(items_hbm.at[pl.ds(base, BPW)], iidx)

    def run(tab, idx, out):
        @pl.loop(0, NCHUNK)
        def _chunks(c):
            off = c * CHUNK
            pltpu.async_copy(tab.at[idx.at[pl.ds(off, CHUNK)]], buf, gsem).wait()
            pltpu.sync_copy(buf, out.at[pl.ds(base + off, CHUNK)])

    run(u_tab, uidx, u_out)
    run(i_tab, iidx, i_out)


def kernel(users, items, user_table, item_table):
    u_repr, i_repr = _gather2(users, items, user_table, item_table)
    return (u_repr, i_repr)


# speedup vs baseline: 1.7696x; 1.1339x over previous
import functools

import jax
import jax.numpy as jnp
from jax import lax
from jax.experimental import pallas as pl
from jax.experimental.pallas import tpu as pltpu
from jax.experimental.pallas import tpu_sc as plsc

BATCH = 4096
DIM = 4096
NUM_CORES = 2
NUM_SUBCORES = 16
NUM_WORKERS = NUM_CORES * NUM_SUBCORES
BPW = BATCH // NUM_WORKERS
CHUNK = 8
NCHUNK = BPW // CHUNK

_MESH = plsc.VectorSubcoreMesh(
    core_axis_name="c", subcore_axis_name="s",
    num_cores=NUM_CORES, num_subcores=NUM_SUBCORES)


@functools.partial(
    pl.kernel,
    out_type=(
        jax.ShapeDtypeStruct((BATCH, DIM), jnp.float32),
        jax.ShapeDtypeStruct((BATCH, DIM), jnp.float32),
    ),
    mesh=_MESH,
    scratch_types=[
        pltpu.VMEM((BPW,), jnp.int32),
        pltpu.VMEM((BPW,), jnp.int32),
        pltpu.VMEM((CHUNK, DIM), jnp.float32),
        pltpu.VMEM((CHUNK, DIM), jnp.float32),
        pltpu.SemaphoreType.DMA,
        pltpu.SemaphoreType.DMA,
        pltpu.SemaphoreType.DMA,
        pltpu.SemaphoreType.DMA,
    ],
)
def _gather2(users_hbm, items_hbm, u_tab, i_tab, u_out, i_out,
             uidx, iidx, buf0, buf1, gsem0, gsem1, ssem0, ssem1):
    wid = lax.axis_index("s") * NUM_CORES + lax.axis_index("c")
    base = wid * BPW
    pltpu.sync_copy(users_hbm.at[pl.ds(base, BPW)], uidx)
    pltpu.sync_copy(items_hbm.at[pl.ds(base, BPW)], iidx)

    def run(tab, idx, out):
        bufs = (buf0, buf1)
        gsems = (gsem0, gsem1)
        ssems = (ssem0, ssem1)

        def gstart(c, b):
            pltpu.async_copy(tab.at[idx.at[pl.ds(c * CHUNK, CHUNK)]],
                             bufs[b], gsems[b])

        def gwait(b):
            pltpu.make_async_copy(tab.at[pl.ds(0, CHUNK)], bufs[b],
                                  gsems[b]).wait()

        def sstart(c, b):
            pltpu.async_copy(bufs[b], out.at[pl.ds(base + c * CHUNK, CHUNK)],
                             ssems[b])

        def swait(b):
            pltpu.make_async_copy(bufs[b], out.at[pl.ds(base, CHUNK)],
                                  ssems[b]).wait()

        gstart(0, 0)
        gstart(1, 1)

        @pl.loop(0, NCHUNK - 2, step=2)
        def _steady(c):
            gwait(0)
            sstart(c, 0)
            gwait(1)
            sstart(c + 1, 1)
            swait(0)
            gstart(c + 2, 0)
            swait(1)
            gstart(c + 3, 1)

        gwait(0)
        sstart(NCHUNK - 2, 0)
        gwait(1)
        sstart(NCHUNK - 1, 1)
        swait(0)
        swait(1)

    run(u_tab, uidx, u_out)
    run(i_tab, iidx, i_out)


def kernel(users, items, user_table, item_table):
    u_repr, i_repr = _gather2(users, items, user_table, item_table)
    return (u_repr, i_repr)
